# baseline (speedup 1.0000x reference)
import math

import jax
import jax.numpy as jnp
from jax import lax
from jax.experimental import pallas as pl
from jax.experimental.pallas import tpu as pltpu
from jax.experimental.pallas import tpu_sc as plsc

NUM_INPUT = 2048
NUM_HIDDEN = 32768
NUM_OUTPUT = 512
N = NUM_INPUT + NUM_HIDDEN + NUM_OUTPUT
FAN_OUT = 32
TAU = 20.0
THRESHOLD = 0.3
HID_START = NUM_INPUT
OUT_START = NUM_INPUT + NUM_HIDDEN
CHK = N - HID_START
OUT_CHK = OUT_START - HID_START
MAX_STEPS = 20
L = 16
NSH = 16
SHARD = CHK // NSH
SH_VECS = SHARD // L
ECH = 128
NSTR = ECH * FAN_OUT // 128
IN_PER = NUM_INPUT // NSH

_INV_DECAY = float(math.exp(1.0 / TAU))


def _snn_body(spk_hbm, tgt_hbm, w_hbm, times_hbm, pots_hbm,
              shard_p, fired_f, front_v, spk_v, idx_v, idx1_v,
              trows_v, trows1_v, wrows_v, wrows1_v,
              eidx_v, eval_v, eidx1_v, eval1_v,
              times_v, cnt_v, counts_all,
              p_sh, counts_sh, sem_g, sem_gw, sem_g1, sem_g1w, sem_s, sem_s1, sem_p, sem_c):
    cid = lax.axis_index("c")
    sid = lax.axis_index("s")

    @pl.when(cid == 0)
    def _():
        my_base = sid * SHARD

        def init_body(i, _):
            sl = pl.ds(i * L, L)
            shard_p[sl] = jnp.zeros((L,), jnp.float32)
            fired_f[sl] = jnp.zeros((L,), jnp.int32)
            front_v[sl] = jnp.zeros((L,), jnp.int32)
            return 0
        lax.fori_loop(0, SH_VECS, init_body, 0)

        def tinit_body(i, _):
            times_v[pl.ds(i * L, L)] = jnp.full((L,), -1, jnp.int32)
            return 0
        lax.fori_loop(0, NUM_OUTPUT // L, tinit_body, 0)

        pltpu.sync_copy(shard_p, p_sh.at[pl.ds(my_base, SHARD)])

        pltpu.sync_copy(spk_hbm.at[pl.ds(sid * IN_PER, IN_PER)], spk_v)

        def in_body(i, off):
            m = spk_v[pl.ds(i * L, L)] > 0
            pcs = plsc.all_reduce_population_count(m)[0]

            @pl.when(pcs > 0)
            def _():
                ids = jax.lax.iota(jnp.int32, L) + (sid * IN_PER + i * L)
                mi = jnp.where(m, jnp.int32(1), jnp.int32(0))
                cs = plsc.cumsum(mi)
                plsc.store_scatter(front_v, [off + cs - 1], ids, mask=m)
            return off + pcs
        cnt0 = lax.fori_loop(0, IN_PER // L, in_body, jnp.int32(0))

        def publish_count(my_cnt):
            cnt_v[pl.ds(0, L)] = jnp.full((L,), my_cnt, jnp.int32)
            return pltpu.async_copy(cnt_v, counts_sh.at[sid], sem_c)

        def read_total():
            pltpu.sync_copy(counts_sh, counts_all)
            tot = jnp.zeros((L,), jnp.int32)
            for j in range(NSH):
                tot = tot + counts_all[j, pl.ds(0, L)]
            return tot[0]

        publish_count(cnt0).wait()
        plsc.subcore_barrier()
        tot0 = read_total()

        def step_cond(carry):
            t, my_cnt, total, g = carry
            return (t < MAX_STEPS) & (total > 0)

        def step_body(carry):
            t, my_cnt, total, g = carry
            amp = jnp.where(t == 0, jnp.float32(2.0) * g, g)
            thr = jnp.float32(THRESHOLD) * g

            def fill_idx(idx_ref, base):
                b = jnp.minimum(base, jnp.maximum(my_cnt - 1, 0))
                b = jnp.minimum(b, jnp.int32(SHARD - ECH))

                def cp_body(j, _):
                    idx_ref[pl.ds(j * L, L)] = front_v[pl.ds(b + j * L, L)]
                    return 0
                lax.fori_loop(0, ECH // L, cp_body, 0)

            def stage_fire(trows_ref, wrows_ref, base, eidx, evals, sem):
                nrows = jnp.maximum(
                    jnp.minimum(jnp.int32(ECH), my_cnt - base), jnp.int32(0))
                rup = ((nrows + 3) // 4) * 4

                def row_body(r, _):
                    valid = r < nrows
                    j = r // 4
                    col = (r % 4) * (2 * L)
                    for h in range(FAN_OUT // L):
                        tv = trows_ref[r, pl.ds(h * L, L)] - HID_START
                        wv = wrows_ref[r, pl.ds(h * L, L)] * amp
                        tv = jnp.where(valid, tv, jnp.int32(0))
                        wv = jnp.where(valid, wv, jnp.float32(0.0))
                        eidx[j, pl.ds(col + h * L, L)] = tv
                        evals[j, pl.ds(col + h * L, L)] = wv
                    return 0
                lax.fori_loop(0, rup, row_body, 0)

                for j in range(NSTR):
                    @pl.when(j * 4 < rup)
                    def _(j=j):
                        pltpu.async_copy(
                            evals.at[j], p_sh.at[eidx.at[j]], sem, add=True)
                return rup

            def drain(eidx, evals, sem, rup):
                for j in range(NSTR):
                    @pl.when(j * 4 < rup)
                    def _(j=j):
                        pltpu.make_async_copy(
                            evals.at[j], p_sh.at[eidx.at[j]], sem).wait()

            nchunks = (my_cnt + (ECH - 1)) // ECH
            nhalf = (nchunks + 1) // 2

            @pl.when(my_cnt > 0)
            def _():
                fill_idx(idx_v, jnp.int32(0))
                pltpu.async_copy(tgt_hbm.at[idx_v], trows_v, sem_g)
                pltpu.async_copy(w_hbm.at[idx_v], wrows_v, sem_gw)

            def pair_body(k, carry):
                rup0p, rup1p = carry
                b0 = (2 * k) * ECH
                b1 = b0 + ECH
                b2 = b1 + ECH
                pltpu.make_async_copy(tgt_hbm.at[idx_v], trows_v, sem_g).wait()
                pltpu.make_async_copy(w_hbm.at[idx_v], wrows_v, sem_gw).wait()

                @pl.when(b1 < my_cnt)
                def _():
                    fill_idx(idx1_v, b1)
                    pltpu.async_copy(tgt_hbm.at[idx1_v], trows1_v, sem_g1)
                    pltpu.async_copy(w_hbm.at[idx1_v], wrows1_v, sem_g1w)
                drain(eidx_v, eval_v, sem_s, rup0p)
                rup0 = stage_fire(trows_v, wrows_v, b0, eidx_v, eval_v, sem_s)

                @pl.when(b2 < my_cnt)
                def _():
                    fill_idx(idx_v, b2)
                    pltpu.async_copy(tgt_hbm.at[idx_v], trows_v, sem_g)
                    pltpu.async_copy(w_hbm.at[idx_v], wrows_v, sem_gw)

                @pl.when(b1 < my_cnt)
                def _():
                    pltpu.make_async_copy(
                        tgt_hbm.at[idx1_v], trows1_v, sem_g1).wait()
                    pltpu.make_async_copy(
                        w_hbm.at[idx1_v], wrows1_v, sem_g1w).wait()
                drain(eidx1_v, eval1_v, sem_s1, rup1p)
                rup1 = stage_fire(trows1_v, wrows1_v, b1,
                                  eidx1_v, eval1_v, sem_s1)
                return rup0, rup1
            rup0f, rup1f = lax.fori_loop(
                0, nhalf, pair_body, (jnp.int32(0), jnp.int32(0)))
            drain(eidx_v, eval_v, sem_s, rup0f)
            drain(eidx1_v, eval1_v, sem_s1, rup1f)
            plsc.subcore_barrier()

            pltpu.sync_copy(p_sh.at[pl.ds(my_base, SHARD)], shard_p)

            UNR = 2

            def sw_body(i2, off):
                for u in range(UNR):
                    i = i2 * UNR + u
                    co = my_base + i * L
                    sl = pl.ds(i * L, L)
                    v = shard_p[sl]
                    f = fired_f[sl]
                    m = (v >= thr) & (f == 0)
                    pcs = plsc.all_reduce_population_count(m)[0]

                    @pl.when(pcs > 0)
                    def _(off=off):
                        mi = jnp.where(m, jnp.int32(1), jnp.int32(0))
                        fired_f[sl] = f | mi
                        keep = m & (co < OUT_CHK)
                        shard_p[sl] = jnp.where(keep, jnp.float32(0.0), v)

                        @pl.when(co >= OUT_CHK)
                        def _():
                            osl = pl.ds(co - OUT_CHK, L)
                            times_v[osl] = jnp.where(m, t, times_v[osl])

                        ids = jax.lax.iota(jnp.int32, L) + (co + HID_START)
                        cs = plsc.cumsum(mi)
                        plsc.store_scatter(
                            front_v, [off + cs - 1], ids, mask=m)
                    off = off + pcs
                return off
            my_new = lax.fori_loop(0, SH_VECS // UNR, sw_body, jnp.int32(0))

            wb = pltpu.async_copy(
                shard_p, p_sh.at[pl.ds(my_base, SHARD)], sem_p)
            pc = publish_count(my_new)
            wb.wait()
            pc.wait()
            plsc.subcore_barrier()
            new_total = read_total()
            return t + 1, my_new, new_total, g * jnp.float32(_INV_DECAY)

        lax.while_loop(step_cond, step_body,
                       (jnp.int32(0), cnt0, tot0, jnp.float32(1.0)))

        @pl.when(sid == NSH - 1)
        def _():
            pltpu.sync_copy(times_v, times_hbm)
            pltpu.sync_copy(
                shard_p.at[pl.ds(OUT_CHK - (NSH - 1) * SHARD, NUM_OUTPUT)],
                pots_hbm)


_snn = pl.kernel(
    _snn_body,
    out_type=(jax.ShapeDtypeStruct((NUM_OUTPUT,), jnp.int32),
              jax.ShapeDtypeStruct((NUM_OUTPUT,), jnp.float32)),
    mesh=plsc.VectorSubcoreMesh(core_axis_name="c", subcore_axis_name="s"),
    compiler_params=pltpu.CompilerParams(needs_layout_passes=False, use_tc_tiling_on_sc=False),
    scratch_types=[
        pltpu.VMEM((SHARD,), jnp.float32),
        pltpu.VMEM((SHARD,), jnp.int32),
        pltpu.VMEM((SHARD,), jnp.int32),
        pltpu.VMEM((IN_PER,), jnp.int32),
        pltpu.VMEM((ECH,), jnp.int32),
        pltpu.VMEM((ECH,), jnp.int32),
        pltpu.VMEM((ECH, FAN_OUT), jnp.int32),
        pltpu.VMEM((ECH, FAN_OUT), jnp.int32),
        pltpu.VMEM((ECH, FAN_OUT), jnp.float32),
        pltpu.VMEM((ECH, FAN_OUT), jnp.float32),
        pltpu.VMEM((NSTR, 128), jnp.int32),
        pltpu.VMEM((NSTR, 128), jnp.float32),
        pltpu.VMEM((NSTR, 128), jnp.int32),
        pltpu.VMEM((NSTR, 128), jnp.float32),
        pltpu.VMEM((NUM_OUTPUT,), jnp.int32),
        pltpu.VMEM((L,), jnp.int32),
        pltpu.VMEM((NSH, L), jnp.int32),
        pltpu.VMEM_SHARED((CHK,), jnp.float32),
        pltpu.VMEM_SHARED((NSH, L), jnp.int32),
        pltpu.SemaphoreType.DMA,
        pltpu.SemaphoreType.DMA,
        pltpu.SemaphoreType.DMA,
        pltpu.SemaphoreType.DMA,
        pltpu.SemaphoreType.DMA,
        pltpu.SemaphoreType.DMA,
        pltpu.SemaphoreType.DMA,
        pltpu.SemaphoreType.DMA,
    ],
)


def kernel(input_spikes, max_timesteps, weights, targets):
    spk = input_spikes.astype(jnp.int32)
    times, pots_scaled = _snn(spk, targets, weights)
    decay_base = jnp.exp(jnp.array(-1.0 / TAU, dtype=jnp.float32))
    scale = decay_base ** jnp.asarray(max_timesteps, jnp.float32)
    return times, pots_scaled * scale


# speedup vs baseline: 1.1044x; 1.1044x over previous
import math

import jax
import jax.numpy as jnp
from jax import lax
from jax.experimental import pallas as pl
from jax.experimental.pallas import tpu as pltpu
from jax.experimental.pallas import tpu_sc as plsc

NUM_INPUT = 2048
NUM_HIDDEN = 32768
NUM_OUTPUT = 512
N = NUM_INPUT + NUM_HIDDEN + NUM_OUTPUT
FAN_OUT = 32
TAU = 20.0
THRESHOLD = 0.3
HID_START = NUM_INPUT
OUT_START = NUM_INPUT + NUM_HIDDEN
CHK = N - HID_START
OUT_CHK = OUT_START - HID_START
MAX_STEPS = 20
L = 16
NSH = 16
SHARD = CHK // NSH
SH_VECS = SHARD // L
ECH = 128
NSTR = ECH * FAN_OUT // 128
IN_PER = NUM_INPUT // NSH

_INV_DECAY = float(math.exp(1.0 / TAU))


def _snn_body(spk_hbm, tgt_hbm, w_hbm, times_hbm, pots_hbm,
              shard_p, fired_f, front_v, spk_v, idx_v, idx1_v,
              trows_v, trows1_v, wrows_v, wrows1_v,
              eidx_v, eval_v, eidx1_v, eval1_v,
              times_v, cnt_v, counts_all,
              p_sh, counts_sh, sem_g, sem_gw, sem_g1, sem_g1w, sem_s, sem_s1, sem_p, sem_c):
    cid = lax.axis_index("c")
    sid = lax.axis_index("s")

    @pl.when(cid == 0)
    def _():
        my_base = sid * SHARD

        def init_body(i, _):
            sl = pl.ds(i * L, L)
            shard_p[sl] = jnp.zeros((L,), jnp.float32)
            fired_f[sl] = jnp.zeros((L,), jnp.int32)
            front_v[sl] = jnp.zeros((L,), jnp.int32)
            return 0
        lax.fori_loop(0, SH_VECS, init_body, 0)

        def tinit_body(i, _):
            times_v[pl.ds(i * L, L)] = jnp.full((L,), -1, jnp.int32)
            return 0
        lax.fori_loop(0, NUM_OUTPUT // L, tinit_body, 0)

        pltpu.sync_copy(shard_p, p_sh.at[pl.ds(my_base, SHARD)])

        pltpu.sync_copy(spk_hbm.at[pl.ds(sid * IN_PER, IN_PER)], spk_v)

        def in_body(i, off):
            m = spk_v[pl.ds(i * L, L)] > 0
            ids = jax.lax.iota(jnp.int32, L) + (sid * IN_PER + i * L)
            mi = jnp.where(m, jnp.int32(1), jnp.int32(0))
            cs = plsc.cumsum(mi)
            plsc.store_scatter(front_v, [off + cs - 1], ids, mask=m)
            return off + cs[L - 1]
        cnt0 = lax.fori_loop(0, IN_PER // L, in_body, jnp.int32(0))

        def publish_count(my_cnt):
            cnt_v[pl.ds(0, L)] = jnp.full((L,), my_cnt, jnp.int32)
            return pltpu.async_copy(cnt_v, counts_sh.at[sid], sem_c)

        def read_total():
            pltpu.sync_copy(counts_sh, counts_all)
            tot = jnp.zeros((L,), jnp.int32)
            for j in range(NSH):
                tot = tot + counts_all[j, pl.ds(0, L)]
            return tot[0]

        publish_count(cnt0).wait()
        plsc.subcore_barrier()
        tot0 = read_total()

        def step_cond(carry):
            t, my_cnt, total, g = carry
            return (t < MAX_STEPS) & (total > 0)

        def step_body(carry):
            t, my_cnt, total, g = carry
            amp = jnp.where(t == 0, jnp.float32(2.0) * g, g)
            thr = jnp.float32(THRESHOLD) * g

            def fill_idx(idx_ref, base):
                b = jnp.minimum(base, jnp.maximum(my_cnt - 1, 0))
                b = jnp.minimum(b, jnp.int32(SHARD - ECH))

                def cp_body(j, _):
                    idx_ref[pl.ds(j * L, L)] = front_v[pl.ds(b + j * L, L)]
                    return 0
                lax.fori_loop(0, ECH // L, cp_body, 0)

            def stage_fire(trows_ref, wrows_ref, base, eidx, evals, sem):
                nrows = jnp.maximum(
                    jnp.minimum(jnp.int32(ECH), my_cnt - base), jnp.int32(0))
                rup = ((nrows + 3) // 4) * 4

                def row_body(r2, _):
                    for u in range(2):
                        r = r2 * 2 + u
                        valid = r < nrows
                        j = r // 4
                        col = (r % 4) * (2 * L)
                        for h in range(FAN_OUT // L):
                            tv = trows_ref[r, pl.ds(h * L, L)] - HID_START
                            wv = wrows_ref[r, pl.ds(h * L, L)] * amp
                            tv = jnp.where(valid, tv, jnp.int32(0))
                            wv = jnp.where(valid, wv, jnp.float32(0.0))
                            eidx[j, pl.ds(col + h * L, L)] = tv
                            evals[j, pl.ds(col + h * L, L)] = wv
                    return 0
                lax.fori_loop(0, rup // 2, row_body, 0)

                for j in range(NSTR):
                    @pl.when(j * 4 < rup)
                    def _(j=j):
                        pltpu.async_copy(
                            evals.at[j], p_sh.at[eidx.at[j]], sem, add=True)
                return rup

            def drain(eidx, evals, sem, rup):
                for j in range(NSTR):
                    @pl.when(j * 4 < rup)
                    def _(j=j):
                        pltpu.make_async_copy(
                            evals.at[j], p_sh.at[eidx.at[j]], sem).wait()

            nchunks = (my_cnt + (ECH - 1)) // ECH
            nhalf = (nchunks + 1) // 2

            @pl.when(my_cnt > 0)
            def _():
                fill_idx(idx_v, jnp.int32(0))
                pltpu.async_copy(tgt_hbm.at[idx_v], trows_v, sem_g)
                pltpu.async_copy(w_hbm.at[idx_v], wrows_v, sem_gw)

            def pair_body(k, carry):
                rup0p, rup1p = carry
                b0 = (2 * k) * ECH
                b1 = b0 + ECH
                b2 = b1 + ECH
                pltpu.make_async_copy(tgt_hbm.at[idx_v], trows_v, sem_g).wait()
                pltpu.make_async_copy(w_hbm.at[idx_v], wrows_v, sem_gw).wait()

                @pl.when(b1 < my_cnt)
                def _():
                    fill_idx(idx1_v, b1)
                    pltpu.async_copy(tgt_hbm.at[idx1_v], trows1_v, sem_g1)
                    pltpu.async_copy(w_hbm.at[idx1_v], wrows1_v, sem_g1w)
                drain(eidx_v, eval_v, sem_s, rup0p)
                rup0 = stage_fire(trows_v, wrows_v, b0, eidx_v, eval_v, sem_s)

                @pl.when(b2 < my_cnt)
                def _():
                    fill_idx(idx_v, b2)
                    pltpu.async_copy(tgt_hbm.at[idx_v], trows_v, sem_g)
                    pltpu.async_copy(w_hbm.at[idx_v], wrows_v, sem_gw)

                @pl.when(b1 < my_cnt)
                def _():
                    pltpu.make_async_copy(
                        tgt_hbm.at[idx1_v], trows1_v, sem_g1).wait()
                    pltpu.make_async_copy(
                        w_hbm.at[idx1_v], wrows1_v, sem_g1w).wait()
                drain(eidx1_v, eval1_v, sem_s1, rup1p)
                rup1 = stage_fire(trows1_v, wrows1_v, b1,
                                  eidx1_v, eval1_v, sem_s1)
                return rup0, rup1
            rup0f, rup1f = lax.fori_loop(
                0, nhalf, pair_body, (jnp.int32(0), jnp.int32(0)))
            drain(eidx_v, eval_v, sem_s, rup0f)
            drain(eidx1_v, eval1_v, sem_s1, rup1f)
            plsc.subcore_barrier()

            pltpu.sync_copy(p_sh.at[pl.ds(my_base, SHARD)], shard_p)

            UNR = 2

            def sw_body(i2, off):
                for u in range(UNR):
                    i = i2 * UNR + u
                    co = my_base + i * L
                    sl = pl.ds(i * L, L)
                    v = shard_p[sl]
                    f = fired_f[sl]
                    m = (v >= thr) & (f == 0)
                    mi = jnp.where(m, jnp.int32(1), jnp.int32(0))
                    fired_f[sl] = f | mi
                    keep = m & (co < OUT_CHK)
                    shard_p[sl] = jnp.where(keep, jnp.float32(0.0), v)

                    @pl.when(co >= OUT_CHK)
                    def _():
                        osl = pl.ds(co - OUT_CHK, L)
                        times_v[osl] = jnp.where(m, t, times_v[osl])

                    ids = jax.lax.iota(jnp.int32, L) + (co + HID_START)
                    cs = plsc.cumsum(mi)
                    plsc.store_scatter(front_v, [off + cs - 1], ids, mask=m)
                    off = off + cs[L - 1]
                return off
            my_new = lax.fori_loop(0, SH_VECS // UNR, sw_body, jnp.int32(0))

            wb = pltpu.async_copy(
                shard_p, p_sh.at[pl.ds(my_base, SHARD)], sem_p)
            pc = publish_count(my_new)
            wb.wait()
            pc.wait()
            plsc.subcore_barrier()
            new_total = read_total()
            return t + 1, my_new, new_total, g * jnp.float32(_INV_DECAY)

        lax.while_loop(step_cond, step_body,
                       (jnp.int32(0), cnt0, tot0, jnp.float32(1.0)))

        @pl.when(sid == NSH - 1)
        def _():
            pltpu.sync_copy(times_v, times_hbm)
            pltpu.sync_copy(
                shard_p.at[pl.ds(OUT_CHK - (NSH - 1) * SHARD, NUM_OUTPUT)],
                pots_hbm)


_snn = pl.kernel(
    _snn_body,
    out_type=(jax.ShapeDtypeStruct((NUM_OUTPUT,), jnp.int32),
              jax.ShapeDtypeStruct((NUM_OUTPUT,), jnp.float32)),
    mesh=plsc.VectorSubcoreMesh(core_axis_name="c", subcore_axis_name="s"),
    compiler_params=pltpu.CompilerParams(needs_layout_passes=False, use_tc_tiling_on_sc=False),
    scratch_types=[
        pltpu.VMEM((SHARD,), jnp.float32),
        pltpu.VMEM((SHARD,), jnp.int32),
        pltpu.VMEM((SHARD,), jnp.int32),
        pltpu.VMEM((IN_PER,), jnp.int32),
        pltpu.VMEM((ECH,), jnp.int32),
        pltpu.VMEM((ECH,), jnp.int32),
        pltpu.VMEM((ECH, FAN_OUT), jnp.int32),
        pltpu.VMEM((ECH, FAN_OUT), jnp.int32),
        pltpu.VMEM((ECH, FAN_OUT), jnp.float32),
        pltpu.VMEM((ECH, FAN_OUT), jnp.float32),
        pltpu.VMEM((NSTR, 128), jnp.int32),
        pltpu.VMEM((NSTR, 128), jnp.float32),
        pltpu.VMEM((NSTR, 128), jnp.int32),
        pltpu.VMEM((NSTR, 128), jnp.float32),
        pltpu.VMEM((NUM_OUTPUT,), jnp.int32),
        pltpu.VMEM((L,), jnp.int32),
        pltpu.VMEM((NSH, L), jnp.int32),
        pltpu.VMEM_SHARED((CHK,), jnp.float32),
        pltpu.VMEM_SHARED((NSH, L), jnp.int32),
        pltpu.SemaphoreType.DMA,
        pltpu.SemaphoreType.DMA,
        pltpu.SemaphoreType.DMA,
        pltpu.SemaphoreType.DMA,
        pltpu.SemaphoreType.DMA,
        pltpu.SemaphoreType.DMA,
        pltpu.SemaphoreType.DMA,
        pltpu.SemaphoreType.DMA,
    ],
)


def kernel(input_spikes, max_timesteps, weights, targets):
    spk = input_spikes.astype(jnp.int32)
    times, pots_scaled = _snn(spk, targets, weights)
    decay_base = jnp.exp(jnp.array(-1.0 / TAU, dtype=jnp.float32))
    scale = decay_base ** jnp.asarray(max_timesteps, jnp.float32)
    return times, pots_scaled * scale


# speedup vs baseline: 1.1068x; 1.0022x over previous
import math

import jax
import jax.numpy as jnp
from jax import lax
from jax.experimental import pallas as pl
from jax.experimental.pallas import tpu as pltpu
from jax.experimental.pallas import tpu_sc as plsc

NUM_INPUT = 2048
NUM_HIDDEN = 32768
NUM_OUTPUT = 512
N = NUM_INPUT + NUM_HIDDEN + NUM_OUTPUT
FAN_OUT = 32
TAU = 20.0
THRESHOLD = 0.3
HID_START = NUM_INPUT
OUT_START = NUM_INPUT + NUM_HIDDEN
CHK = N - HID_START
OUT_CHK = OUT_START - HID_START
MAX_STEPS = 20
L = 16
NSH = 16
SHARD = CHK // NSH
SH_VECS = SHARD // L
ECH = 128
NSTR = ECH * FAN_OUT // 128
IN_PER = NUM_INPUT // NSH

_INV_DECAY = float(math.exp(1.0 / TAU))


def _snn_body(spk_hbm, tgt_hbm, w_hbm, times_hbm, pots_hbm,
              shard_p, fired_f, front_v, spk_v, idx_v, idx1_v,
              trows_v, trows1_v, wrows_v, wrows1_v,
              eidx_v, eval_v, eidx1_v, eval1_v,
              times_v, cnt_v, counts_all,
              p_sh, counts_sh, sem_g, sem_gw, sem_g1, sem_g1w, sem_s, sem_s1, sem_p, sem_c):
    cid = lax.axis_index("c")
    sid = lax.axis_index("s")

    @pl.when(cid == 0)
    def _():
        my_base = sid * SHARD

        def init_body(i, _):
            sl = pl.ds(i * L, L)
            shard_p[sl] = jnp.zeros((L,), jnp.float32)
            fired_f[sl] = jnp.zeros((L,), jnp.int32)
            front_v[sl] = jnp.zeros((L,), jnp.int32)
            return 0
        lax.fori_loop(0, SH_VECS, init_body, 0)

        def tinit_body(i, _):
            times_v[pl.ds(i * L, L)] = jnp.full((L,), -1, jnp.int32)
            return 0
        lax.fori_loop(0, NUM_OUTPUT // L, tinit_body, 0)

        pltpu.sync_copy(shard_p, p_sh.at[pl.ds(my_base, SHARD)])

        pltpu.sync_copy(spk_hbm.at[pl.ds(sid * IN_PER, IN_PER)], spk_v)

        def in_body(i, off):
            m = spk_v[pl.ds(i * L, L)] > 0
            ids = jax.lax.iota(jnp.int32, L) + (sid * IN_PER + i * L)
            mi = jnp.where(m, jnp.int32(1), jnp.int32(0))
            cs = plsc.cumsum(mi)
            plsc.store_scatter(front_v, [off + cs - 1], ids, mask=m)
            return off + cs[L - 1]
        cnt0 = lax.fori_loop(0, IN_PER // L, in_body, jnp.int32(0))

        def publish_count(my_cnt):
            cnt_v[pl.ds(0, L)] = jnp.full((L,), my_cnt, jnp.int32)
            return pltpu.async_copy(cnt_v, counts_sh.at[sid], sem_c)

        def read_total():
            pltpu.sync_copy(counts_sh, counts_all)
            tot = jnp.zeros((L,), jnp.int32)
            for j in range(NSH):
                tot = tot + counts_all[j, pl.ds(0, L)]
            return tot[0]

        publish_count(cnt0).wait()
        plsc.subcore_barrier()
        tot0 = read_total()

        def step_cond(carry):
            t, my_cnt, total, g = carry
            return (t < MAX_STEPS) & (total > 0)

        def step_body(carry):
            t, my_cnt, total, g = carry
            amp = jnp.where(t == 0, jnp.float32(2.0) * g, g)
            thr = jnp.float32(THRESHOLD) * g

            def fill_idx(idx_ref, base):
                b = jnp.minimum(base, jnp.maximum(my_cnt - 1, 0))
                b = jnp.minimum(b, jnp.int32(SHARD - ECH))

                def cp_body(j, _):
                    idx_ref[pl.ds(j * L, L)] = front_v[pl.ds(b + j * L, L)]
                    return 0
                lax.fori_loop(0, ECH // L, cp_body, 0)

            def stage_fire(trows_ref, wrows_ref, base, eidx, evals, sem):
                nrows = jnp.maximum(
                    jnp.minimum(jnp.int32(ECH), my_cnt - base), jnp.int32(0))
                rup = ((nrows + 3) // 4) * 4

                def row_body(r, _):
                    valid = r < nrows
                    j = r // 4
                    col = (r % 4) * (2 * L)
                    for h in range(FAN_OUT // L):
                        tv = trows_ref[r, pl.ds(h * L, L)] - HID_START
                        wv = wrows_ref[r, pl.ds(h * L, L)] * amp
                        tv = jnp.where(valid, tv, jnp.int32(0))
                        wv = jnp.where(valid, wv, jnp.float32(0.0))
                        eidx[j, pl.ds(col + h * L, L)] = tv
                        evals[j, pl.ds(col + h * L, L)] = wv
                    return 0
                lax.fori_loop(0, rup, row_body, 0)

                for j in range(NSTR):
                    @pl.when(j * 4 < rup)
                    def _(j=j):
                        pltpu.async_copy(
                            evals.at[j], p_sh.at[eidx.at[j]], sem, add=True)
                return rup

            def drain(eidx, evals, sem, rup):
                for j in range(NSTR):
                    @pl.when(j * 4 < rup)
                    def _(j=j):
                        pltpu.make_async_copy(
                            evals.at[j], p_sh.at[eidx.at[j]], sem).wait()

            nchunks = (my_cnt + (ECH - 1)) // ECH
            nhalf = (nchunks + 1) // 2

            @pl.when(my_cnt > 0)
            def _():
                fill_idx(idx_v, jnp.int32(0))
                pltpu.async_copy(tgt_hbm.at[idx_v], trows_v, sem_g)
                pltpu.async_copy(w_hbm.at[idx_v], wrows_v, sem_gw)

            def pair_body(k, carry):
                rup0p, rup1p = carry
                b0 = (2 * k) * ECH
                b1 = b0 + ECH
                b2 = b1 + ECH
                pltpu.make_async_copy(tgt_hbm.at[idx_v], trows_v, sem_g).wait()
                pltpu.make_async_copy(w_hbm.at[idx_v], wrows_v, sem_gw).wait()

                @pl.when(b1 < my_cnt)
                def _():
                    fill_idx(idx1_v, b1)
                    pltpu.async_copy(tgt_hbm.at[idx1_v], trows1_v, sem_g1)
                    pltpu.async_copy(w_hbm.at[idx1_v], wrows1_v, sem_g1w)
                drain(eidx_v, eval_v, sem_s, rup0p)
                rup0 = stage_fire(trows_v, wrows_v, b0, eidx_v, eval_v, sem_s)

                @pl.when(b2 < my_cnt)
                def _():
                    fill_idx(idx_v, b2)
                    pltpu.async_copy(tgt_hbm.at[idx_v], trows_v, sem_g)
                    pltpu.async_copy(w_hbm.at[idx_v], wrows_v, sem_gw)

                @pl.when(b1 < my_cnt)
                def _():
                    pltpu.make_async_copy(
                        tgt_hbm.at[idx1_v], trows1_v, sem_g1).wait()
                    pltpu.make_async_copy(
                        w_hbm.at[idx1_v], wrows1_v, sem_g1w).wait()
                drain(eidx1_v, eval1_v, sem_s1, rup1p)
                rup1 = stage_fire(trows1_v, wrows1_v, b1,
                                  eidx1_v, eval1_v, sem_s1)
                return rup0, rup1
            rup0f, rup1f = lax.fori_loop(
                0, nhalf, pair_body, (jnp.int32(0), jnp.int32(0)))
            drain(eidx_v, eval_v, sem_s, rup0f)
            drain(eidx1_v, eval1_v, sem_s1, rup1f)
            plsc.subcore_barrier()

            pltpu.sync_copy(p_sh.at[pl.ds(my_base, SHARD)], shard_p)

            UNR = 2

            def sw_body(i2, off):
                for u in range(UNR):
                    i = i2 * UNR + u
                    co = my_base + i * L
                    sl = pl.ds(i * L, L)
                    v = shard_p[sl]
                    f = fired_f[sl]
                    m = (v >= thr) & (f == 0)
                    mi = jnp.where(m, jnp.int32(1), jnp.int32(0))
                    fired_f[sl] = f | mi
                    keep = m & (co < OUT_CHK)
                    shard_p[sl] = jnp.where(keep, jnp.float32(0.0), v)

                    @pl.when(co >= OUT_CHK)
                    def _():
                        osl = pl.ds(co - OUT_CHK, L)
                        times_v[osl] = jnp.where(m, t, times_v[osl])

                    ids = jax.lax.iota(jnp.int32, L) + (co + HID_START)
                    cs = plsc.cumsum(mi)
                    plsc.store_scatter(front_v, [off + cs - 1], ids, mask=m)
                    off = off + cs[L - 1]
                return off
            my_new = lax.fori_loop(0, SH_VECS // UNR, sw_body, jnp.int32(0))

            wb = pltpu.async_copy(
                shard_p, p_sh.at[pl.ds(my_base, SHARD)], sem_p)
            pc = publish_count(my_new)
            wb.wait()
            pc.wait()
            plsc.subcore_barrier()
            new_total = read_total()
            return t + 1, my_new, new_total, g * jnp.float32(_INV_DECAY)

        lax.while_loop(step_cond, step_body,
                       (jnp.int32(0), cnt0, tot0, jnp.float32(1.0)))

        @pl.when(sid == NSH - 1)
        def _():
            pltpu.sync_copy(times_v, times_hbm)
            pltpu.sync_copy(
                shard_p.at[pl.ds(OUT_CHK - (NSH - 1) * SHARD, NUM_OUTPUT)],
                pots_hbm)


_snn = pl.kernel(
    _snn_body,
    out_type=(jax.ShapeDtypeStruct((NUM_OUTPUT,), jnp.int32),
              jax.ShapeDtypeStruct((NUM_OUTPUT,), jnp.float32)),
    mesh=plsc.VectorSubcoreMesh(core_axis_name="c", subcore_axis_name="s"),
    compiler_params=pltpu.CompilerParams(needs_layout_passes=False, use_tc_tiling_on_sc=False),
    scratch_types=[
        pltpu.VMEM((SHARD,), jnp.float32),
        pltpu.VMEM((SHARD,), jnp.int32),
        pltpu.VMEM((SHARD,), jnp.int32),
        pltpu.VMEM((IN_PER,), jnp.int32),
        pltpu.VMEM((ECH,), jnp.int32),
        pltpu.VMEM((ECH,), jnp.int32),
        pltpu.VMEM((ECH, FAN_OUT), jnp.int32),
        pltpu.VMEM((ECH, FAN_OUT), jnp.int32),
        pltpu.VMEM((ECH, FAN_OUT), jnp.float32),
        pltpu.VMEM((ECH, FAN_OUT), jnp.float32),
        pltpu.VMEM((NSTR, 128), jnp.int32),
        pltpu.VMEM((NSTR, 128), jnp.float32),
        pltpu.VMEM((NSTR, 128), jnp.int32),
        pltpu.VMEM((NSTR, 128), jnp.float32),
        pltpu.VMEM((NUM_OUTPUT,), jnp.int32),
        pltpu.VMEM((L,), jnp.int32),
        pltpu.VMEM((NSH, L), jnp.int32),
        pltpu.VMEM_SHARED((CHK,), jnp.float32),
        pltpu.VMEM_SHARED((NSH, L), jnp.int32),
        pltpu.SemaphoreType.DMA,
        pltpu.SemaphoreType.DMA,
        pltpu.SemaphoreType.DMA,
        pltpu.SemaphoreType.DMA,
        pltpu.SemaphoreType.DMA,
        pltpu.SemaphoreType.DMA,
        pltpu.SemaphoreType.DMA,
        pltpu.SemaphoreType.DMA,
    ],
)


def kernel(input_spikes, max_timesteps, weights, targets):
    spk = input_spikes.astype(jnp.int32)
    times, pots_scaled = _snn(spk, targets, weights)
    decay_base = jnp.exp(jnp.array(-1.0 / TAU, dtype=jnp.float32))
    scale = decay_base ** jnp.asarray(max_timesteps, jnp.float32)
    return times, pots_scaled * scale


# speedup vs baseline: 1.2703x; 1.1478x over previous
import math

import jax
import jax.numpy as jnp
from jax import lax
from jax.experimental import pallas as pl
from jax.experimental.pallas import tpu as pltpu
from jax.experimental.pallas import tpu_sc as plsc

NUM_INPUT = 2048
NUM_HIDDEN = 32768
NUM_OUTPUT = 512
N = NUM_INPUT + NUM_HIDDEN + NUM_OUTPUT
FAN_OUT = 32
TAU = 20.0
THRESHOLD = 0.3
HID_START = NUM_INPUT
OUT_START = NUM_INPUT + NUM_HIDDEN
CHK = N - HID_START
OUT_CHK = OUT_START - HID_START
MAX_STEPS = 20
L = 16
NSH = 16
SHARD = CHK // NSH
SH_VECS = SHARD // L
ECH = 64
NSTR = ECH * FAN_OUT // 128
IN_PER = NUM_INPUT // NSH

_INV_DECAY = float(math.exp(1.0 / TAU))


def _snn_body(spk_hbm, tgt_hbm, w_hbm, times_hbm, pots_hbm,
              shard_p, fired_f, front_v, spk_v, idx_v, idx1_v,
              trows_v, trows1_v, wrows_v, wrows1_v,
              eidx_v, eval_v, eidx1_v, eval1_v,
              times_v, cnt_v, counts_all,
              p_sh, counts_sh, sem_g, sem_gw, sem_g1, sem_g1w, sem_s, sem_s1, sem_p, sem_c):
    cid = lax.axis_index("c")
    sid = lax.axis_index("s")

    @pl.when(cid == 0)
    def _():
        my_base = sid * SHARD

        def init_body(i, _):
            sl = pl.ds(i * L, L)
            shard_p[sl] = jnp.zeros((L,), jnp.float32)
            fired_f[sl] = jnp.zeros((L,), jnp.int32)
            front_v[sl] = jnp.zeros((L,), jnp.int32)
            return 0
        lax.fori_loop(0, SH_VECS, init_body, 0)

        def tinit_body(i, _):
            times_v[pl.ds(i * L, L)] = jnp.full((L,), -1, jnp.int32)
            return 0
        lax.fori_loop(0, NUM_OUTPUT // L, tinit_body, 0)

        pltpu.sync_copy(shard_p, p_sh.at[pl.ds(my_base, SHARD)])

        pltpu.sync_copy(spk_hbm.at[pl.ds(sid * IN_PER, IN_PER)], spk_v)

        def in_body(i, off):
            m = spk_v[pl.ds(i * L, L)] > 0
            ids = jax.lax.iota(jnp.int32, L) + (sid * IN_PER + i * L)
            mi = jnp.where(m, jnp.int32(1), jnp.int32(0))
            cs = plsc.cumsum(mi)
            plsc.store_scatter(front_v, [off + cs - 1], ids, mask=m)
            return off + cs[L - 1]
        cnt0 = lax.fori_loop(0, IN_PER // L, in_body, jnp.int32(0))

        def publish_count(my_cnt):
            cnt_v[pl.ds(0, L)] = jnp.full((L,), my_cnt, jnp.int32)
            return pltpu.async_copy(cnt_v, counts_sh.at[sid], sem_c)

        def read_total():
            pltpu.sync_copy(counts_sh, counts_all)
            tot = jnp.zeros((L,), jnp.int32)
            for j in range(NSH):
                tot = tot + counts_all[j, pl.ds(0, L)]
            return tot[0]

        publish_count(cnt0).wait()
        plsc.subcore_barrier()
        tot0 = read_total()

        def step_cond(carry):
            t, my_cnt, total, g = carry
            return (t < MAX_STEPS) & (total > 0)

        def step_body(carry):
            t, my_cnt, total, g = carry
            amp = jnp.where(t == 0, jnp.float32(2.0) * g, g)
            thr = jnp.float32(THRESHOLD) * g

            def fill_idx(idx_ref, base):
                b = jnp.minimum(base, jnp.maximum(my_cnt - 1, 0))
                b = jnp.minimum(b, jnp.int32(SHARD - ECH))

                def cp_body(j, _):
                    idx_ref[pl.ds(j * L, L)] = front_v[pl.ds(b + j * L, L)]
                    return 0
                lax.fori_loop(0, ECH // L, cp_body, 0)

            def stage_fire(trows_ref, wrows_ref, base, eidx, evals, sem):
                nrows = jnp.maximum(
                    jnp.minimum(jnp.int32(ECH), my_cnt - base), jnp.int32(0))
                rup = ((nrows + 3) // 4) * 4

                def row_body(r, _):
                    valid = r < nrows
                    j = r // 4
                    col = (r % 4) * (2 * L)
                    for h in range(FAN_OUT // L):
                        tv = trows_ref[r, pl.ds(h * L, L)] - HID_START
                        wv = wrows_ref[r, pl.ds(h * L, L)] * amp
                        tv = jnp.where(valid, tv, jnp.int32(0))
                        wv = jnp.where(valid, wv, jnp.float32(0.0))
                        eidx[j, pl.ds(col + h * L, L)] = tv
                        evals[j, pl.ds(col + h * L, L)] = wv
                    return 0
                lax.fori_loop(0, rup, row_body, 0)

                for j in range(NSTR):
                    @pl.when(j * 4 < rup)
                    def _(j=j):
                        pltpu.async_copy(
                            evals.at[j], p_sh.at[eidx.at[j]], sem, add=True)
                return rup

            def drain(eidx, evals, sem, rup):
                for j in range(NSTR):
                    @pl.when(j * 4 < rup)
                    def _(j=j):
                        pltpu.make_async_copy(
                            evals.at[j], p_sh.at[eidx.at[j]], sem).wait()

            nchunks = (my_cnt + (ECH - 1)) // ECH
            nhalf = (nchunks + 1) // 2

            @pl.when(my_cnt > 0)
            def _():
                fill_idx(idx_v, jnp.int32(0))
                pltpu.async_copy(tgt_hbm.at[idx_v], trows_v, sem_g)
                pltpu.async_copy(w_hbm.at[idx_v], wrows_v, sem_gw)

            def pair_body(k, carry):
                rup0p, rup1p = carry
                b0 = (2 * k) * ECH
                b1 = b0 + ECH
                b2 = b1 + ECH
                pltpu.make_async_copy(tgt_hbm.at[idx_v], trows_v, sem_g).wait()
                pltpu.make_async_copy(w_hbm.at[idx_v], wrows_v, sem_gw).wait()

                @pl.when(b1 < my_cnt)
                def _():
                    fill_idx(idx1_v, b1)
                    pltpu.async_copy(tgt_hbm.at[idx1_v], trows1_v, sem_g1)
                    pltpu.async_copy(w_hbm.at[idx1_v], wrows1_v, sem_g1w)
                drain(eidx_v, eval_v, sem_s, rup0p)
                rup0 = stage_fire(trows_v, wrows_v, b0, eidx_v, eval_v, sem_s)

                @pl.when(b2 < my_cnt)
                def _():
                    fill_idx(idx_v, b2)
                    pltpu.async_copy(tgt_hbm.at[idx_v], trows_v, sem_g)
                    pltpu.async_copy(w_hbm.at[idx_v], wrows_v, sem_gw)

                @pl.when(b1 < my_cnt)
                def _():
                    pltpu.make_async_copy(
                        tgt_hbm.at[idx1_v], trows1_v, sem_g1).wait()
                    pltpu.make_async_copy(
                        w_hbm.at[idx1_v], wrows1_v, sem_g1w).wait()
                drain(eidx1_v, eval1_v, sem_s1, rup1p)
                rup1 = stage_fire(trows1_v, wrows1_v, b1,
                                  eidx1_v, eval1_v, sem_s1)
                return rup0, rup1
            rup0f, rup1f = lax.fori_loop(
                0, nhalf, pair_body, (jnp.int32(0), jnp.int32(0)))
            drain(eidx_v, eval_v, sem_s, rup0f)
            drain(eidx1_v, eval1_v, sem_s1, rup1f)
            plsc.subcore_barrier()

            pltpu.sync_copy(p_sh.at[pl.ds(my_base, SHARD)], shard_p)

            UNR = 2

            def sw_body(i2, off):
                for u in range(UNR):
                    i = i2 * UNR + u
                    co = my_base + i * L
                    sl = pl.ds(i * L, L)
                    v = shard_p[sl]
                    f = fired_f[sl]
                    m = (v >= thr) & (f == 0)
                    mi = jnp.where(m, jnp.int32(1), jnp.int32(0))
                    fired_f[sl] = f | mi
                    keep = m & (co < OUT_CHK)
                    shard_p[sl] = jnp.where(keep, jnp.float32(0.0), v)

                    @pl.when(co >= OUT_CHK)
                    def _():
                        osl = pl.ds(co - OUT_CHK, L)
                        times_v[osl] = jnp.where(m, t, times_v[osl])

                    ids = jax.lax.iota(jnp.int32, L) + (co + HID_START)
                    cs = plsc.cumsum(mi)
                    plsc.store_scatter(front_v, [off + cs - 1], ids, mask=m)
                    off = off + cs[L - 1]
                return off
            my_new = lax.fori_loop(0, SH_VECS // UNR, sw_body, jnp.int32(0))

            wb = pltpu.async_copy(
                shard_p, p_sh.at[pl.ds(my_base, SHARD)], sem_p)
            pc = publish_count(my_new)
            wb.wait()
            pc.wait()
            plsc.subcore_barrier()
            new_total = read_total()
            return t + 1, my_new, new_total, g * jnp.float32(_INV_DECAY)

        lax.while_loop(step_cond, step_body,
                       (jnp.int32(0), cnt0, tot0, jnp.float32(1.0)))

        @pl.when(sid == NSH - 1)
        def _():
            pltpu.sync_copy(times_v, times_hbm)
            pltpu.sync_copy(
                shard_p.at[pl.ds(OUT_CHK - (NSH - 1) * SHARD, NUM_OUTPUT)],
                pots_hbm)


_snn = pl.kernel(
    _snn_body,
    out_type=(jax.ShapeDtypeStruct((NUM_OUTPUT,), jnp.int32),
              jax.ShapeDtypeStruct((NUM_OUTPUT,), jnp.float32)),
    mesh=plsc.VectorSubcoreMesh(core_axis_name="c", subcore_axis_name="s"),
    compiler_params=pltpu.CompilerParams(needs_layout_passes=False, use_tc_tiling_on_sc=False),
    scratch_types=[
        pltpu.VMEM((SHARD,), jnp.float32),
        pltpu.VMEM((SHARD,), jnp.int32),
        pltpu.VMEM((SHARD,), jnp.int32),
        pltpu.VMEM((IN_PER,), jnp.int32),
        pltpu.VMEM((ECH,), jnp.int32),
        pltpu.VMEM((ECH,), jnp.int32),
        pltpu.VMEM((ECH, FAN_OUT), jnp.int32),
        pltpu.VMEM((ECH, FAN_OUT), jnp.int32),
        pltpu.VMEM((ECH, FAN_OUT), jnp.float32),
        pltpu.VMEM((ECH, FAN_OUT), jnp.float32),
        pltpu.VMEM((NSTR, 128), jnp.int32),
        pltpu.VMEM((NSTR, 128), jnp.float32),
        pltpu.VMEM((NSTR, 128), jnp.int32),
        pltpu.VMEM((NSTR, 128), jnp.float32),
        pltpu.VMEM((NUM_OUTPUT,), jnp.int32),
        pltpu.VMEM((L,), jnp.int32),
        pltpu.VMEM((NSH, L), jnp.int32),
        pltpu.VMEM_SHARED((CHK,), jnp.float32),
        pltpu.VMEM_SHARED((NSH, L), jnp.int32),
        pltpu.SemaphoreType.DMA,
        pltpu.SemaphoreType.DMA,
        pltpu.SemaphoreType.DMA,
        pltpu.SemaphoreType.DMA,
        pltpu.SemaphoreType.DMA,
        pltpu.SemaphoreType.DMA,
        pltpu.SemaphoreType.DMA,
        pltpu.SemaphoreType.DMA,
    ],
)


def kernel(input_spikes, max_timesteps, weights, targets):
    spk = input_spikes.astype(jnp.int32)
    times, pots_scaled = _snn(spk, targets, weights)
    decay_base = jnp.exp(jnp.array(-1.0 / TAU, dtype=jnp.float32))
    scale = decay_base ** jnp.asarray(max_timesteps, jnp.float32)
    return times, pots_scaled * scale


# speedup vs baseline: 1.3044x; 1.0269x over previous
import math

import jax
import jax.numpy as jnp
from jax import lax
from jax.experimental import pallas as pl
from jax.experimental.pallas import tpu as pltpu
from jax.experimental.pallas import tpu_sc as plsc

NUM_INPUT = 2048
NUM_HIDDEN = 32768
NUM_OUTPUT = 512
N = NUM_INPUT + NUM_HIDDEN + NUM_OUTPUT
FAN_OUT = 32
TAU = 20.0
THRESHOLD = 0.3
HID_START = NUM_INPUT
OUT_START = NUM_INPUT + NUM_HIDDEN
CHK = N - HID_START
OUT_CHK = OUT_START - HID_START
MAX_STEPS = 20
L = 16
NSH = 16
SHARD = CHK // NSH
SH_VECS = SHARD // L
ECH = 32
NSTR = ECH * FAN_OUT // 128
IN_PER = NUM_INPUT // NSH

_INV_DECAY = float(math.exp(1.0 / TAU))


def _snn_body(spk_hbm, tgt_hbm, w_hbm, times_hbm, pots_hbm,
              shard_p, fired_f, front_v, spk_v, idx_v, idx1_v,
              trows_v, trows1_v, wrows_v, wrows1_v,
              eidx_v, eval_v, eidx1_v, eval1_v,
              times_v, cnt_v, counts_all,
              p_sh, counts_sh, sem_g, sem_gw, sem_g1, sem_g1w, sem_s, sem_s1, sem_p, sem_c):
    cid = lax.axis_index("c")
    sid = lax.axis_index("s")

    @pl.when(cid == 0)
    def _():
        my_base = sid * SHARD

        def init_body(i, _):
            sl = pl.ds(i * L, L)
            shard_p[sl] = jnp.zeros((L,), jnp.float32)
            fired_f[sl] = jnp.zeros((L,), jnp.int32)
            front_v[sl] = jnp.zeros((L,), jnp.int32)
            return 0
        lax.fori_loop(0, SH_VECS, init_body, 0)

        def tinit_body(i, _):
            times_v[pl.ds(i * L, L)] = jnp.full((L,), -1, jnp.int32)
            return 0
        lax.fori_loop(0, NUM_OUTPUT // L, tinit_body, 0)

        pltpu.sync_copy(shard_p, p_sh.at[pl.ds(my_base, SHARD)])

        pltpu.sync_copy(spk_hbm.at[pl.ds(sid * IN_PER, IN_PER)], spk_v)

        def in_body(i, off):
            m = spk_v[pl.ds(i * L, L)] > 0
            ids = jax.lax.iota(jnp.int32, L) + (sid * IN_PER + i * L)
            mi = jnp.where(m, jnp.int32(1), jnp.int32(0))
            cs = plsc.cumsum(mi)
            plsc.store_scatter(front_v, [off + cs - 1], ids, mask=m)
            return off + cs[L - 1]
        cnt0 = lax.fori_loop(0, IN_PER // L, in_body, jnp.int32(0))

        def publish_count(my_cnt):
            cnt_v[pl.ds(0, L)] = jnp.full((L,), my_cnt, jnp.int32)
            return pltpu.async_copy(cnt_v, counts_sh.at[sid], sem_c)

        def read_total():
            pltpu.sync_copy(counts_sh, counts_all)
            tot = jnp.zeros((L,), jnp.int32)
            for j in range(NSH):
                tot = tot + counts_all[j, pl.ds(0, L)]
            return tot[0]

        publish_count(cnt0).wait()
        plsc.subcore_barrier()
        tot0 = read_total()

        def step_cond(carry):
            t, my_cnt, total, g = carry
            return (t < MAX_STEPS) & (total > 0)

        def step_body(carry):
            t, my_cnt, total, g = carry
            amp = jnp.where(t == 0, jnp.float32(2.0) * g, g)
            thr = jnp.float32(THRESHOLD) * g

            def fill_idx(idx_ref, base):
                b = jnp.minimum(base, jnp.maximum(my_cnt - 1, 0))
                b = jnp.minimum(b, jnp.int32(SHARD - ECH))

                def cp_body(j, _):
                    idx_ref[pl.ds(j * L, L)] = front_v[pl.ds(b + j * L, L)]
                    return 0
                lax.fori_loop(0, ECH // L, cp_body, 0)

            def stage_fire(trows_ref, wrows_ref, base, eidx, evals, sem):
                nrows = jnp.maximum(
                    jnp.minimum(jnp.int32(ECH), my_cnt - base), jnp.int32(0))
                rup = ((nrows + 3) // 4) * 4

                def row_body(r, _):
                    valid = r < nrows
                    j = r // 4
                    col = (r % 4) * (2 * L)
                    for h in range(FAN_OUT // L):
                        tv = trows_ref[r, pl.ds(h * L, L)] - HID_START
                        wv = wrows_ref[r, pl.ds(h * L, L)] * amp
                        tv = jnp.where(valid, tv, jnp.int32(0))
                        wv = jnp.where(valid, wv, jnp.float32(0.0))
                        eidx[j, pl.ds(col + h * L, L)] = tv
                        evals[j, pl.ds(col + h * L, L)] = wv
                    return 0
                lax.fori_loop(0, rup, row_body, 0)

                for j in range(NSTR):
                    @pl.when(j * 4 < rup)
                    def _(j=j):
                        pltpu.async_copy(
                            evals.at[j], p_sh.at[eidx.at[j]], sem, add=True)
                return rup

            def drain(eidx, evals, sem, rup):
                for j in range(NSTR):
                    @pl.when(j * 4 < rup)
                    def _(j=j):
                        pltpu.make_async_copy(
                            evals.at[j], p_sh.at[eidx.at[j]], sem).wait()

            nchunks = (my_cnt + (ECH - 1)) // ECH
            nhalf = (nchunks + 1) // 2

            @pl.when(my_cnt > 0)
            def _():
                fill_idx(idx_v, jnp.int32(0))
                pltpu.async_copy(tgt_hbm.at[idx_v], trows_v, sem_g)
                pltpu.async_copy(w_hbm.at[idx_v], wrows_v, sem_gw)

            def pair_body(k, carry):
                rup0p, rup1p = carry
                b0 = (2 * k) * ECH
                b1 = b0 + ECH
                b2 = b1 + ECH
                pltpu.make_async_copy(tgt_hbm.at[idx_v], trows_v, sem_g).wait()
                pltpu.make_async_copy(w_hbm.at[idx_v], wrows_v, sem_gw).wait()

                @pl.when(b1 < my_cnt)
                def _():
                    fill_idx(idx1_v, b1)
                    pltpu.async_copy(tgt_hbm.at[idx1_v], trows1_v, sem_g1)
                    pltpu.async_copy(w_hbm.at[idx1_v], wrows1_v, sem_g1w)
                drain(eidx_v, eval_v, sem_s, rup0p)
                rup0 = stage_fire(trows_v, wrows_v, b0, eidx_v, eval_v, sem_s)

                @pl.when(b2 < my_cnt)
                def _():
                    fill_idx(idx_v, b2)
                    pltpu.async_copy(tgt_hbm.at[idx_v], trows_v, sem_g)
                    pltpu.async_copy(w_hbm.at[idx_v], wrows_v, sem_gw)

                @pl.when(b1 < my_cnt)
                def _():
                    pltpu.make_async_copy(
                        tgt_hbm.at[idx1_v], trows1_v, sem_g1).wait()
                    pltpu.make_async_copy(
                        w_hbm.at[idx1_v], wrows1_v, sem_g1w).wait()
                drain(eidx1_v, eval1_v, sem_s1, rup1p)
                rup1 = stage_fire(trows1_v, wrows1_v, b1,
                                  eidx1_v, eval1_v, sem_s1)
                return rup0, rup1
            rup0f, rup1f = lax.fori_loop(
                0, nhalf, pair_body, (jnp.int32(0), jnp.int32(0)))
            drain(eidx_v, eval_v, sem_s, rup0f)
            drain(eidx1_v, eval1_v, sem_s1, rup1f)
            plsc.subcore_barrier()

            pltpu.sync_copy(p_sh.at[pl.ds(my_base, SHARD)], shard_p)

            UNR = 2

            def sw_body(i2, off):
                for u in range(UNR):
                    i = i2 * UNR + u
                    co = my_base + i * L
                    sl = pl.ds(i * L, L)
                    v = shard_p[sl]
                    f = fired_f[sl]
                    m = (v >= thr) & (f == 0)
                    mi = jnp.where(m, jnp.int32(1), jnp.int32(0))
                    fired_f[sl] = f | mi
                    keep = m & (co < OUT_CHK)
                    shard_p[sl] = jnp.where(keep, jnp.float32(0.0), v)

                    @pl.when(co >= OUT_CHK)
                    def _():
                        osl = pl.ds(co - OUT_CHK, L)
                        times_v[osl] = jnp.where(m, t, times_v[osl])

                    ids = jax.lax.iota(jnp.int32, L) + (co + HID_START)
                    cs = plsc.cumsum(mi)
                    plsc.store_scatter(front_v, [off + cs - 1], ids, mask=m)
                    off = off + cs[L - 1]
                return off
            my_new = lax.fori_loop(0, SH_VECS // UNR, sw_body, jnp.int32(0))

            wb = pltpu.async_copy(
                shard_p, p_sh.at[pl.ds(my_base, SHARD)], sem_p)
            pc = publish_count(my_new)
            wb.wait()
            pc.wait()
            plsc.subcore_barrier()
            new_total = read_total()
            return t + 1, my_new, new_total, g * jnp.float32(_INV_DECAY)

        lax.while_loop(step_cond, step_body,
                       (jnp.int32(0), cnt0, tot0, jnp.float32(1.0)))

        @pl.when(sid == NSH - 1)
        def _():
            pltpu.sync_copy(times_v, times_hbm)
            pltpu.sync_copy(
                shard_p.at[pl.ds(OUT_CHK - (NSH - 1) * SHARD, NUM_OUTPUT)],
                pots_hbm)


_snn = pl.kernel(
    _snn_body,
    out_type=(jax.ShapeDtypeStruct((NUM_OUTPUT,), jnp.int32),
              jax.ShapeDtypeStruct((NUM_OUTPUT,), jnp.float32)),
    mesh=plsc.VectorSubcoreMesh(core_axis_name="c", subcore_axis_name="s"),
    compiler_params=pltpu.CompilerParams(needs_layout_passes=False, use_tc_tiling_on_sc=False),
    scratch_types=[
        pltpu.VMEM((SHARD,), jnp.float32),
        pltpu.VMEM((SHARD,), jnp.int32),
        pltpu.VMEM((SHARD,), jnp.int32),
        pltpu.VMEM((IN_PER,), jnp.int32),
        pltpu.VMEM((ECH,), jnp.int32),
        pltpu.VMEM((ECH,), jnp.int32),
        pltpu.VMEM((ECH, FAN_OUT), jnp.int32),
        pltpu.VMEM((ECH, FAN_OUT), jnp.int32),
        pltpu.VMEM((ECH, FAN_OUT), jnp.float32),
        pltpu.VMEM((ECH, FAN_OUT), jnp.float32),
        pltpu.VMEM((NSTR, 128), jnp.int32),
        pltpu.VMEM((NSTR, 128), jnp.float32),
        pltpu.VMEM((NSTR, 128), jnp.int32),
        pltpu.VMEM((NSTR, 128), jnp.float32),
        pltpu.VMEM((NUM_OUTPUT,), jnp.int32),
        pltpu.VMEM((L,), jnp.int32),
        pltpu.VMEM((NSH, L), jnp.int32),
        pltpu.VMEM_SHARED((CHK,), jnp.float32),
        pltpu.VMEM_SHARED((NSH, L), jnp.int32),
        pltpu.SemaphoreType.DMA,
        pltpu.SemaphoreType.DMA,
        pltpu.SemaphoreType.DMA,
        pltpu.SemaphoreType.DMA,
        pltpu.SemaphoreType.DMA,
        pltpu.SemaphoreType.DMA,
        pltpu.SemaphoreType.DMA,
        pltpu.SemaphoreType.DMA,
    ],
)


def kernel(input_spikes, max_timesteps, weights, targets):
    spk = input_spikes.astype(jnp.int32)
    times, pots_scaled = _snn(spk, targets, weights)
    decay_base = jnp.exp(jnp.array(-1.0 / TAU, dtype=jnp.float32))
    scale = decay_base ** jnp.asarray(max_timesteps, jnp.float32)
    return times, pots_scaled * scale
